# Initial kernel scaffold; baseline (speedup 1.0000x reference)
import jax
import jax.numpy as jnp
from jax.experimental import pallas as pl


def kernel(x):
    raise NotImplementedError("write your pallas kernel here")


# speedup vs baseline: 2.1824x; 2.1824x over previous
import functools

import numpy as np
import jax
import jax.numpy as jnp
from jax import lax
from jax.experimental import pallas as pl
from jax.experimental.pallas import tpu as pltpu
from jax.experimental.pallas import tpu_sc as plsc

_N = 70
_NX = _N * (_N + 1) // 2
_OUT = _N * _N

_info = plsc.get_sparse_core_info()
_NC, _NS, _L = _info.num_cores, _info.num_subcores, _info.num_lanes
_NW = _NC * _NS
_PAD = ((_OUT + 16 * _NW - 1) // (16 * _NW)) * (16 * _NW)
_CHUNK = _PAD // _NW
_NV = _CHUNK // 16


def _build_tables():
    i = np.arange(_N)[:, None]
    j = np.arange(_N)[None, :]
    r = np.maximum(i, j)
    c = np.minimum(i, j)
    idx = (r * (r + 1) // 2 + c).astype(np.int32).reshape(-1)
    scale = np.where(i == j, 1.0, np.sqrt(0.5)).astype(np.float32).reshape(-1)
    idx_p = np.zeros((_PAD,), np.int32)
    scale_p = np.zeros((_PAD,), np.float32)
    idx_p[:_OUT] = idx
    scale_p[:_OUT] = scale
    return idx_p, scale_p


_IDX_TAB, _SCALE_TAB = _build_tables()


@functools.partial(
    pl.kernel,
    mesh=plsc.VectorSubcoreMesh(core_axis_name="c", subcore_axis_name="s"),
    out_type=jax.ShapeDtypeStruct((_PAD,), jnp.float32),
    scratch_types=[
        pltpu.VMEM((_NX,), jnp.float32),
        pltpu.VMEM((_CHUNK,), jnp.int32),
        pltpu.VMEM((_CHUNK,), jnp.float32),
        pltpu.VMEM((_CHUNK,), jnp.float32),
    ],
    compiler_params=pltpu.CompilerParams(needs_layout_passes=False),
)
def _smat_sc(x_hbm, idx_hbm, scale_hbm, out_hbm, x_v, idx_v, scale_v, o_v):
    wid = lax.axis_index("s") * _NC + lax.axis_index("c")
    base = wid * _CHUNK
    pltpu.sync_copy(x_hbm, x_v)
    pltpu.sync_copy(idx_hbm.at[pl.ds(base, _CHUNK)], idx_v)
    pltpu.sync_copy(scale_hbm.at[pl.ds(base, _CHUNK)], scale_v)
    for v in range(_NV):
        sl = pl.ds(v * 16, 16)
        vals = plsc.load_gather(x_v, [idx_v[sl]])
        o_v[sl] = vals * scale_v[sl]
    pltpu.sync_copy(o_v, out_hbm.at[pl.ds(base, _CHUNK)])


def kernel(x):
    out_flat = _smat_sc(x, _IDX_TAB, _SCALE_TAB)
    return out_flat[:_OUT].reshape(_N, _N)


# speedup vs baseline: 2.2768x; 1.0432x over previous
import functools

import numpy as np
import jax
import jax.numpy as jnp
from jax import lax
from jax.experimental import pallas as pl
from jax.experimental.pallas import tpu as pltpu
from jax.experimental.pallas import tpu_sc as plsc

_N = 70
_NX = _N * (_N + 1) // 2
_OUT = _N * _N

_info = plsc.get_sparse_core_info()
_NC, _NS, _L = _info.num_cores, _info.num_subcores, _info.num_lanes
_NW = _NC * _NS
_PAD = ((_OUT + 16 * _NW - 1) // (16 * _NW)) * (16 * _NW)
_CHUNK = _PAD // _NW
_NV = _CHUNK // 16


_C_HALF = float(np.sqrt(np.float32(0.5)))


@functools.partial(
    pl.kernel,
    mesh=plsc.VectorSubcoreMesh(core_axis_name="c", subcore_axis_name="s"),
    out_type=jax.ShapeDtypeStruct((_PAD,), jnp.float32),
    scratch_types=[
        pltpu.VMEM((_NX,), jnp.float32),
        pltpu.VMEM((_CHUNK,), jnp.float32),
    ],
    compiler_params=pltpu.CompilerParams(needs_layout_passes=False),
)
def _smat_sc(x_hbm, out_hbm, x_v, o_v):
    wid = lax.axis_index("s") * _NC + lax.axis_index("c")
    base = wid * _CHUNK
    pltpu.sync_copy(x_hbm, x_v)
    lane = lax.iota(jnp.int32, 16)
    pos0 = base + lane
    for v in range(_NV):
        pos = pos0 + (v * 16)
        i = pos // _N
        j = pos - i * _N
        r = jnp.maximum(i, j)
        c = jnp.minimum(i, j)
        idx = jnp.minimum((r * (r + 1)) // 2 + c, _NX - 1)
        vals = plsc.load_gather(x_v, [idx])
        scl = jnp.where(i == j, jnp.float32(1.0), jnp.float32(_C_HALF))
        o_v[pl.ds(v * 16, 16)] = vals * scl
    pltpu.sync_copy(o_v, out_hbm.at[pl.ds(base, _CHUNK)])


def kernel(x):
    out_flat = _smat_sc(x)
    return out_flat[:_OUT].reshape(_N, _N)


# speedup vs baseline: 2.4234x; 1.0644x over previous
import functools

import numpy as np
import jax
import jax.numpy as jnp
from jax import lax
from jax.experimental import pallas as pl
from jax.experimental.pallas import tpu as pltpu
from jax.experimental.pallas import tpu_sc as plsc

_N = 70
_NX = _N * (_N + 1) // 2
_OUT = _N * _N

_info = plsc.get_sparse_core_info()
_NC, _NS, _L = _info.num_cores, _info.num_subcores, _info.num_lanes
_NC = 1
_NW = _NC * _NS
_PAD = ((_OUT + 16 * _NW - 1) // (16 * _NW)) * (16 * _NW)
_CHUNK = _PAD // _NW
_NV = _CHUNK // 16


_C_HALF = float(np.sqrt(np.float32(0.5)))


@functools.partial(
    pl.kernel,
    mesh=plsc.VectorSubcoreMesh(core_axis_name="c", subcore_axis_name="s",
                                num_cores=_NC),
    out_type=jax.ShapeDtypeStruct((_PAD,), jnp.float32),
    scratch_types=[
        pltpu.VMEM((_NX,), jnp.float32),
        pltpu.VMEM((_CHUNK,), jnp.float32),
    ],
    compiler_params=pltpu.CompilerParams(needs_layout_passes=False),
)
def _smat_sc(x_hbm, out_hbm, x_v, o_v):
    wid = lax.axis_index("s") * _NC + lax.axis_index("c")
    base = wid * _CHUNK
    pltpu.sync_copy(x_hbm, x_v)
    lane = lax.iota(jnp.int32, 16)
    pos0 = base + lane
    for v in range(_NV):
        pos = pos0 + (v * 16)
        i = pos // _N
        j = pos - i * _N
        r = jnp.maximum(i, j)
        c = jnp.minimum(i, j)
        idx = jnp.minimum((r * (r + 1)) // 2 + c, _NX - 1)
        vals = plsc.load_gather(x_v, [idx])
        scl = jnp.where(i == j, jnp.float32(1.0), jnp.float32(_C_HALF))
        o_v[pl.ds(v * 16, 16)] = vals * scl
    pltpu.sync_copy(o_v, out_hbm.at[pl.ds(base, _CHUNK)])


def kernel(x):
    out_flat = _smat_sc(x)
    return out_flat[:_OUT].reshape(_N, _N)
